# baseline (speedup 1.0000x reference)
import functools

import jax
import jax.numpy as jnp
from jax import lax
from jax.experimental import pallas as pl
from jax.experimental.pallas import tpu as pltpu
from jax.experimental.pallas import tpu_sc as plsc

N = 10000
E = 320000
D = 128
G = 64
N_CLS = 10

NPAD = 10240
NBLK = NPAD // 128
NC = 2
NS = 16
NW = NC * NS
CHUNK = 128
NCH = 80
NBUF = 2
EPW = NCH * CHUNK
EPAD = NW * EPW
ROWS_PER_TILE = NPAD // NS


def _zero_vmem_rows(ref, nrows):
    z = jnp.zeros((16,), jnp.float32)

    @pl.loop(0, nrows)
    def _(i):
        for k in range(8):
            ref[i, pl.ds(k * 16, 16)] = z


def _sc_count_body(ei_hbm, out_hbm, dst_v, ones_v, stage_v, cnt_sh, sem):
    cid = lax.axis_index("c")
    sid = lax.axis_index("s")
    wid = cid * NS + sid

    one = jnp.ones((16,), jnp.float32)
    zero = jnp.zeros((16,), jnp.float32)

    @pl.loop(0, CHUNK // 16)
    def _(i):
        ones_v[pl.ds(i * 16, 16)] = one

    @pl.loop(0, ROWS_PER_TILE // 16)
    def _(i):
        stage_v[pl.ds(i * 16, 16)] = zero

    pltpu.sync_copy(stage_v, cnt_sh.at[pl.ds(sid * ROWS_PER_TILE, ROWS_PER_TILE)])

    pltpu.sync_copy(ei_hbm.at[wid], dst_v)
    plsc.subcore_barrier()

    @pl.loop(0, NCH)
    def _(j):
        pltpu.sync_copy(ones_v, cnt_sh.at[dst_v.at[j]], add=True)

    plsc.subcore_barrier()
    pltpu.sync_copy(cnt_sh.at[pl.ds(sid * ROWS_PER_TILE, ROWS_PER_TILE)], stage_v)
    pltpu.sync_copy(stage_v, out_hbm.at[cid, pl.ds(sid * ROWS_PER_TILE, ROWS_PER_TILE)])


@functools.cache
def _get_sc_count():
    mesh = plsc.VectorSubcoreMesh(core_axis_name="c", subcore_axis_name="s")
    return pl.kernel(
        _sc_count_body,
        out_type=jax.ShapeDtypeStruct((NC, NPAD), jnp.float32),
        mesh=mesh,
        scratch_types=[
            pltpu.VMEM((NCH, CHUNK), jnp.int32),
            pltpu.VMEM((CHUNK,), jnp.float32),
            pltpu.VMEM((ROWS_PER_TILE,), jnp.float32),
            pltpu.VMEM_SHARED((NPAD,), jnp.float32),
            pltpu.SemaphoreType.DMA,
        ],
    )


def _sc_scatter_body(src_hbm, dst_hbm, xs_hbm, out_hbm, dst_v, src_big,
                     r0, acc_sh, g0):
    cid = lax.axis_index("c")
    sid = lax.axis_index("s")
    wid = cid * NS + sid

    _zero_vmem_rows(r0, CHUNK)

    base = sid * ROWS_PER_TILE
    for t in range(ROWS_PER_TILE // CHUNK):
        pltpu.sync_copy(r0, acc_sh.at[pl.ds(base + t * CHUNK, CHUNK)])

    pltpu.sync_copy(dst_hbm.at[wid], dst_v)
    pltpu.sync_copy(src_hbm.at[wid], src_big)
    plsc.subcore_barrier()

    @pl.loop(0, NCH)
    def _(j):
        pltpu.async_copy(xs_hbm.at[src_big.at[j]], r0, g0).wait()
        pltpu.sync_copy(r0, acc_sh.at[dst_v.at[j]], add=True)

    plsc.subcore_barrier()
    for t in range(ROWS_PER_TILE // CHUNK):
        r = base + t * CHUNK
        pltpu.sync_copy(acc_sh.at[pl.ds(r, CHUNK)], r0)
        pltpu.sync_copy(r0, out_hbm.at[cid, pl.ds(r, CHUNK)])


@functools.cache
def _get_sc_scatter():
    mesh = plsc.VectorSubcoreMesh(core_axis_name="c", subcore_axis_name="s")
    return pl.kernel(
        _sc_scatter_body,
        out_type=jax.ShapeDtypeStruct((NC, NPAD, D), jnp.float32),
        mesh=mesh,
        scratch_types=[
            pltpu.VMEM((NCH, CHUNK), jnp.int32),
            pltpu.VMEM((NCH, CHUNK), jnp.int32),
            pltpu.VMEM((CHUNK, D), jnp.float32),
            pltpu.VMEM_SHARED((NPAD, D), jnp.float32),
            pltpu.SemaphoreType.DMA,
        ],
    )


def _tc_first_body(cnt_ref, x_ref, w_ref, xs_ref, dinv_ref):
    i = pl.program_id(0)
    cnt = cnt_ref[0, 0, 0, :] + cnt_ref[1, 0, 0, :] + 1.0
    row = i * 128 + lax.broadcasted_iota(jnp.int32, (128,), 0)
    dinv = jnp.where(row < N, lax.rsqrt(cnt), 0.0)
    dinv_ref[0, 0, :] = dinv
    xw = jnp.dot(x_ref[...], w_ref[...], preferred_element_type=jnp.float32)
    xs_ref[...] = xw * dinv[:, None]


def _tc_first(cnt, xpad, W1):
    cnt3 = cnt.reshape(NC, NBLK, 1, 128)
    return pl.pallas_call(
        _tc_first_body,
        grid=(NBLK,),
        in_specs=[
            pl.BlockSpec((NC, 1, 1, 128), lambda i: (0, i, 0, 0)),
            pl.BlockSpec((128, D), lambda i: (i, 0)),
            pl.BlockSpec((D, D), lambda i: (0, 0)),
        ],
        out_specs=[
            pl.BlockSpec((128, D), lambda i: (i, 0)),
            pl.BlockSpec((1, 1, 128), lambda i: (i, 0, 0)),
        ],
        out_shape=[
            jax.ShapeDtypeStruct((NPAD, D), jnp.float32),
            jax.ShapeDtypeStruct((NBLK, 1, 128), jnp.float32),
        ],
    )(cnt3, xpad, W1)


def _tc_mid_body(acc_ref, xs_ref, dinv_ref, b_ref, w_ref, xs2_ref):
    dinv = dinv_ref[0, 0, :][:, None]
    h = acc_ref[0] + acc_ref[1] + xs_ref[...]
    h = jax.nn.relu(h * dinv + b_ref[0, :][None, :])
    xw = jnp.dot(h, w_ref[...], preferred_element_type=jnp.float32)
    xs2_ref[...] = xw * dinv


def _tc_mid(acc, xs, dinv, b, W):
    return pl.pallas_call(
        _tc_mid_body,
        grid=(NBLK,),
        in_specs=[
            pl.BlockSpec((NC, 128, D), lambda i: (0, i, 0)),
            pl.BlockSpec((128, D), lambda i: (i, 0)),
            pl.BlockSpec((1, 1, 128), lambda i: (i, 0, 0)),
            pl.BlockSpec((1, D), lambda i: (0, 0)),
            pl.BlockSpec((D, D), lambda i: (0, 0)),
        ],
        out_specs=pl.BlockSpec((128, D), lambda i: (i, 0)),
        out_shape=jax.ShapeDtypeStruct((NPAD, D), jnp.float32),
    )(acc, xs, dinv, b.reshape(1, D), W)


def _tc_last_body(acc_ref, xs_ref, dinv_ref, b_ref, batch_ref,
                  wl1_ref, bl1_ref, wl2_ref, bl2_ref, out_ref,
                  pool_ref, cnt_ref):
    i = pl.program_id(0)

    @pl.when(i == 0)
    def _():
        pool_ref[...] = jnp.zeros((G, D), jnp.float32)
        cnt_ref[...] = jnp.zeros((G, 128), jnp.float32)

    dinv = dinv_ref[0, 0, :][:, None]
    h = acc_ref[0] + acc_ref[1] + xs_ref[...]
    h = jax.nn.relu(h * dinv + b_ref[0, :][None, :])
    ids = batch_ref[0, 0, :]
    p = (ids[:, None] == lax.broadcasted_iota(jnp.int32, (128, G), 1)).astype(
        jnp.float32
    )
    pool_ref[...] += jnp.dot(p.T, h, preferred_element_type=jnp.float32)
    cnt_ref[...] += jnp.dot(
        p.T, jnp.ones((128, 128), jnp.float32), preferred_element_type=jnp.float32
    )

    @pl.when(i == pl.num_programs(0) - 1)
    def _():
        cnt = jnp.maximum(cnt_ref[:, 0:1], 1.0)
        g = pool_ref[...] / cnt
        g = jax.nn.relu(
            jnp.dot(g, wl1_ref[...], preferred_element_type=jnp.float32)
            + bl1_ref[0, :][None, :]
        )
        out_ref[...] = (
            jnp.dot(g, wl2_ref[...], preferred_element_type=jnp.float32)
            + bl2_ref[0, :][None, :]
        )


def _tc_last(acc, xs, dinv, b, batch3, Wl1, bl1, Wl2p, bl2p):
    return pl.pallas_call(
        _tc_last_body,
        grid=(NBLK,),
        in_specs=[
            pl.BlockSpec((NC, 128, D), lambda i: (0, i, 0)),
            pl.BlockSpec((128, D), lambda i: (i, 0)),
            pl.BlockSpec((1, 1, 128), lambda i: (i, 0, 0)),
            pl.BlockSpec((1, D), lambda i: (0, 0)),
            pl.BlockSpec((1, 1, 128), lambda i: (i, 0, 0)),
            pl.BlockSpec((D, D), lambda i: (0, 0)),
            pl.BlockSpec((1, D), lambda i: (0, 0)),
            pl.BlockSpec((D, 128), lambda i: (0, 0)),
            pl.BlockSpec((1, 128), lambda i: (0, 0)),
        ],
        out_specs=pl.BlockSpec((G, 128), lambda i: (0, 0)),
        out_shape=jax.ShapeDtypeStruct((G, 128), jnp.float32),
        scratch_shapes=[
            pltpu.VMEM((G, D), jnp.float32),
            pltpu.VMEM((G, 128), jnp.float32),
        ],
    )(acc, xs, dinv, b.reshape(1, D), batch3, Wl1, bl1.reshape(1, D), Wl2p, bl2p)


def kernel(x, edge_index, batch, W1, b1, W2, b2, Wl1, bl1, Wl2, bl2):
    xpad = jnp.pad(x, ((0, NPAD - N), (0, 0)))
    ei = edge_index.astype(jnp.int32)
    ei = jnp.pad(ei, ((0, 0), (0, EPAD - E)), constant_values=N)
    ei4 = ei.reshape(2, NW, NCH, CHUNK)
    batch3 = jnp.pad(batch.astype(jnp.int32), (0, NPAD - N), constant_values=-1)
    batch3 = batch3.reshape(NBLK, 1, 128)
    Wl2p = jnp.pad(Wl2, ((0, 0), (0, 128 - N_CLS)))
    bl2p = jnp.pad(bl2, (0, 128 - N_CLS)).reshape(1, 128)

    sc_count = _get_sc_count()
    sc_scatter = _get_sc_scatter()
    cnt = sc_count(ei4[1])
    xs1, dinv = _tc_first(cnt, xpad, W1)

    acc1 = sc_scatter(ei4[0], ei4[1], xs1)
    xs2 = _tc_mid(acc1, xs1, dinv, b1, W2)
    acc2 = sc_scatter(ei4[0], ei4[1], xs2)
    out = _tc_last(acc2, xs2, dinv, b2, batch3, Wl1, bl1, Wl2p, bl2p)
    return out[:, :N_CLS]


# speedup vs baseline: 1.5456x; 1.5456x over previous
import functools

import jax
import jax.numpy as jnp
from jax import lax
from jax.experimental import pallas as pl
from jax.experimental.pallas import tpu as pltpu
from jax.experimental.pallas import tpu_sc as plsc

N = 10000
E = 320000
D = 128
G = 64
N_CLS = 10

NPAD = 10240
NBLK = NPAD // 128
NC = 2
NS = 16
NW = NC * NS
CHUNK = 128
NCH = 79
NBUF = 2
EPW = NCH * CHUNK
EPAD = NW * EPW
ROWS_PER_TILE = NPAD // NS


def _zero_vmem_rows(ref, nrows):
    z = jnp.zeros((16,), jnp.float32)

    @pl.loop(0, nrows)
    def _(i):
        for k in range(8):
            ref[i, pl.ds(k * 16, 16)] = z


def _sc_count_body(ei_hbm, out_hbm, dst_v, ones_v, stage_v, cnt_sh, sem):
    cid = lax.axis_index("c")
    sid = lax.axis_index("s")
    wid = cid * NS + sid

    one = jnp.ones((16,), jnp.float32)
    zero = jnp.zeros((16,), jnp.float32)

    @pl.loop(0, CHUNK // 16)
    def _(i):
        ones_v[pl.ds(i * 16, 16)] = one

    @pl.loop(0, ROWS_PER_TILE // 16)
    def _(i):
        stage_v[pl.ds(i * 16, 16)] = zero

    pltpu.sync_copy(stage_v, cnt_sh.at[pl.ds(sid * ROWS_PER_TILE, ROWS_PER_TILE)])

    pltpu.sync_copy(ei_hbm.at[1, wid], dst_v)
    plsc.subcore_barrier()

    @pl.loop(0, NCH)
    def _(j):
        pltpu.sync_copy(ones_v, cnt_sh.at[dst_v.at[j]], add=True)

    plsc.subcore_barrier()
    pltpu.sync_copy(cnt_sh.at[pl.ds(sid * ROWS_PER_TILE, ROWS_PER_TILE)], stage_v)
    pltpu.sync_copy(stage_v, out_hbm.at[cid, pl.ds(sid * ROWS_PER_TILE, ROWS_PER_TILE)])


@functools.cache
def _get_sc_count():
    mesh = plsc.VectorSubcoreMesh(core_axis_name="c", subcore_axis_name="s")
    return pl.kernel(
        _sc_count_body,
        out_type=jax.ShapeDtypeStruct((NC, NPAD), jnp.float32),
        mesh=mesh,
        scratch_types=[
            pltpu.VMEM((NCH, CHUNK), jnp.int32),
            pltpu.VMEM((CHUNK,), jnp.float32),
            pltpu.VMEM((ROWS_PER_TILE,), jnp.float32),
            pltpu.VMEM_SHARED((NPAD,), jnp.float32),
            pltpu.SemaphoreType.DMA,
        ],
    )


def _sc_scatter_body(ei_hbm, xs_hbm, out_hbm, src_v, dst_v,
                     r0, acc_sh, g0):
    cid = lax.axis_index("c")
    sid = lax.axis_index("s")
    wid = cid * NS + sid

    _zero_vmem_rows(r0, CHUNK)

    base = sid * ROWS_PER_TILE
    for t in range(ROWS_PER_TILE // CHUNK):
        pltpu.sync_copy(r0, acc_sh.at[pl.ds(base + t * CHUNK, CHUNK)])

    pltpu.sync_copy(ei_hbm.at[0, wid], src_v)
    pltpu.sync_copy(ei_hbm.at[1, wid], dst_v)
    plsc.subcore_barrier()

    @pl.loop(0, NCH)
    def _(j):
        pltpu.async_copy(xs_hbm.at[src_v.at[j]], r0, g0).wait()
        pltpu.sync_copy(r0, acc_sh.at[dst_v.at[j]], add=True)

    plsc.subcore_barrier()
    for t in range(ROWS_PER_TILE // CHUNK):
        r = base + t * CHUNK
        pltpu.sync_copy(acc_sh.at[pl.ds(r, CHUNK)], r0)
        pltpu.sync_copy(r0, out_hbm.at[cid, pl.ds(r, CHUNK)])


@functools.cache
def _get_sc_scatter():
    mesh = plsc.VectorSubcoreMesh(core_axis_name="c", subcore_axis_name="s")
    return pl.kernel(
        _sc_scatter_body,
        out_type=jax.ShapeDtypeStruct((NC, NPAD, D), jnp.float32),
        mesh=mesh,
        scratch_types=[
            pltpu.VMEM((NCH, CHUNK), jnp.int32),
            pltpu.VMEM((NCH, CHUNK), jnp.int32),
            pltpu.VMEM((CHUNK, D), jnp.float32),
            pltpu.VMEM_SHARED((NPAD, D), jnp.float32),
            pltpu.SemaphoreType.DMA,
        ],
    )


def _tc_first_body(cnt_ref, x_ref, w_ref, xs_ref, dinv_ref):
    i = pl.program_id(0)
    cnt = cnt_ref[0, 0, 0, :] + cnt_ref[1, 0, 0, :] + 1.0
    row = i * 128 + lax.broadcasted_iota(jnp.int32, (128,), 0)
    dinv = jnp.where(row < N, lax.rsqrt(cnt), 0.0)
    dinv_ref[0, 0, :] = dinv
    xw = jnp.dot(x_ref[...], w_ref[...], preferred_element_type=jnp.float32)
    xs_ref[...] = xw * dinv[:, None]


def _tc_first(cnt, xpad, W1):
    cnt3 = cnt.reshape(NC, NBLK, 1, 128)
    return pl.pallas_call(
        _tc_first_body,
        grid=(NBLK,),
        in_specs=[
            pl.BlockSpec((NC, 1, 1, 128), lambda i: (0, i, 0, 0)),
            pl.BlockSpec((128, D), lambda i: (i, 0)),
            pl.BlockSpec((D, D), lambda i: (0, 0)),
        ],
        out_specs=[
            pl.BlockSpec((128, D), lambda i: (i, 0)),
            pl.BlockSpec((1, 1, 128), lambda i: (i, 0, 0)),
        ],
        out_shape=[
            jax.ShapeDtypeStruct((NPAD, D), jnp.float32),
            jax.ShapeDtypeStruct((NBLK, 1, 128), jnp.float32),
        ],
    )(cnt3, xpad, W1)


def _tc_mid_body(acc_ref, xs_ref, dinv_ref, b_ref, w_ref, xs2_ref):
    dinv = dinv_ref[0, 0, :][:, None]
    h = acc_ref[0] + acc_ref[1] + xs_ref[...]
    h = jax.nn.relu(h * dinv + b_ref[0, :][None, :])
    xw = jnp.dot(h, w_ref[...], preferred_element_type=jnp.float32)
    xs2_ref[...] = xw * dinv


def _tc_mid(acc, xs, dinv, b, W):
    return pl.pallas_call(
        _tc_mid_body,
        grid=(NBLK,),
        in_specs=[
            pl.BlockSpec((NC, 128, D), lambda i: (0, i, 0)),
            pl.BlockSpec((128, D), lambda i: (i, 0)),
            pl.BlockSpec((1, 1, 128), lambda i: (i, 0, 0)),
            pl.BlockSpec((1, D), lambda i: (0, 0)),
            pl.BlockSpec((D, D), lambda i: (0, 0)),
        ],
        out_specs=pl.BlockSpec((128, D), lambda i: (i, 0)),
        out_shape=jax.ShapeDtypeStruct((NPAD, D), jnp.float32),
    )(acc, xs, dinv, b.reshape(1, D), W)


def _tc_last_body(acc_ref, xs_ref, dinv_ref, b_ref, batch_ref,
                  wl1_ref, bl1_ref, wl2_ref, bl2_ref, out_ref,
                  pool_ref, cnt_ref):
    i = pl.program_id(0)

    @pl.when(i == 0)
    def _():
        pool_ref[...] = jnp.zeros((G, D), jnp.float32)
        cnt_ref[...] = jnp.zeros((G, 128), jnp.float32)

    dinv = dinv_ref[0, 0, :][:, None]
    h = acc_ref[0] + acc_ref[1] + xs_ref[...]
    h = jax.nn.relu(h * dinv + b_ref[0, :][None, :])
    ids = batch_ref[0, 0, :]
    p = (ids[:, None] == lax.broadcasted_iota(jnp.int32, (128, G), 1)).astype(
        jnp.float32
    )
    pool_ref[...] += jnp.dot(p.T, h, preferred_element_type=jnp.float32)
    cnt_ref[...] += jnp.dot(
        p.T, jnp.ones((128, 128), jnp.float32), preferred_element_type=jnp.float32
    )

    @pl.when(i == pl.num_programs(0) - 1)
    def _():
        cnt = jnp.maximum(cnt_ref[:, 0:1], 1.0)
        g = pool_ref[...] / cnt
        g = jax.nn.relu(
            jnp.dot(g, wl1_ref[...], preferred_element_type=jnp.float32)
            + bl1_ref[0, :][None, :]
        )
        out_ref[...] = (
            jnp.dot(g, wl2_ref[...], preferred_element_type=jnp.float32)
            + bl2_ref[0, :][None, :]
        )


def _tc_last(acc, xs, dinv, b, batch3, Wl1, bl1, Wl2p, bl2p):
    return pl.pallas_call(
        _tc_last_body,
        grid=(NBLK,),
        in_specs=[
            pl.BlockSpec((NC, 128, D), lambda i: (0, i, 0)),
            pl.BlockSpec((128, D), lambda i: (i, 0)),
            pl.BlockSpec((1, 1, 128), lambda i: (i, 0, 0)),
            pl.BlockSpec((1, D), lambda i: (0, 0)),
            pl.BlockSpec((1, 1, 128), lambda i: (i, 0, 0)),
            pl.BlockSpec((D, D), lambda i: (0, 0)),
            pl.BlockSpec((1, D), lambda i: (0, 0)),
            pl.BlockSpec((D, 128), lambda i: (0, 0)),
            pl.BlockSpec((1, 128), lambda i: (0, 0)),
        ],
        out_specs=pl.BlockSpec((G, 128), lambda i: (0, 0)),
        out_shape=jax.ShapeDtypeStruct((G, 128), jnp.float32),
        scratch_shapes=[
            pltpu.VMEM((G, D), jnp.float32),
            pltpu.VMEM((G, 128), jnp.float32),
        ],
    )(acc, xs, dinv, b.reshape(1, D), batch3, Wl1, bl1.reshape(1, D), Wl2p, bl2p)


def kernel(x, edge_index, batch, W1, b1, W2, b2, Wl1, bl1, Wl2, bl2):
    xpad = jnp.pad(x, ((0, NPAD - N), (0, 0)))
    ei = edge_index.astype(jnp.int32)
    ei = jnp.pad(ei, ((0, 0), (0, EPAD - E)), constant_values=N)
    ei4 = ei.reshape(2, NW, NCH, CHUNK)
    batch3 = jnp.pad(batch.astype(jnp.int32), (0, NPAD - N), constant_values=-1)
    batch3 = batch3.reshape(NBLK, 1, 128)
    Wl2p = jnp.pad(Wl2, ((0, 0), (0, 128 - N_CLS)))
    bl2p = jnp.pad(bl2, (0, 128 - N_CLS)).reshape(1, 128)

    sc_count = _get_sc_count()
    sc_scatter = _get_sc_scatter()
    cnt = sc_count(ei4)
    xs1, dinv = _tc_first(cnt, xpad, W1)

    acc1 = sc_scatter(ei4, xs1)
    xs2 = _tc_mid(acc1, xs1, dinv, b1, W2)
    acc2 = sc_scatter(ei4, xs2)
    out = _tc_last(acc2, xs2, dinv, b2, batch3, Wl1, bl1, Wl2p, bl2p)
    return out[:, :N_CLS]


# speedup vs baseline: 2.3709x; 1.5339x over previous
import functools

import jax
import jax.numpy as jnp
from jax import lax
from jax.experimental import pallas as pl
from jax.experimental.pallas import tpu as pltpu
from jax.experimental.pallas import tpu_sc as plsc

N = 10000
E = 320000
D = 128
G = 64
N_CLS = 10

NPAD = 10240
NBLK = NPAD // 128
NC = 2
NS = 16
NW = NC * NS
CHUNK = 128
NCH = 79
NBUF = 2
EPW = NCH * CHUNK
EPAD = NW * EPW
ROWS_PER_TILE = NPAD // NS


def _zero_vmem_rows(ref, nrows):
    z = jnp.zeros((16,), jnp.float32)

    @pl.loop(0, nrows)
    def _(i):
        for k in range(8):
            ref[i, pl.ds(k * 16, 16)] = z


def _sc_count_body(ei_hbm, out_hbm, dst_v, ones_v, stage_v, cnt_sh, sem):
    cid = lax.axis_index("c")
    sid = lax.axis_index("s")
    wid = cid * NS + sid

    one = jnp.ones((16,), jnp.float32)
    zero = jnp.zeros((16,), jnp.float32)

    @pl.loop(0, CHUNK // 16)
    def _(i):
        ones_v[pl.ds(i * 16, 16)] = one

    @pl.loop(0, ROWS_PER_TILE // 16)
    def _(i):
        stage_v[pl.ds(i * 16, 16)] = zero

    pltpu.sync_copy(stage_v, cnt_sh.at[pl.ds(sid * ROWS_PER_TILE, ROWS_PER_TILE)])

    pltpu.sync_copy(ei_hbm.at[1, wid], dst_v)
    plsc.subcore_barrier()

    @pl.loop(0, NCH)
    def _(j):
        pltpu.sync_copy(ones_v, cnt_sh.at[dst_v.at[j]], add=True)

    plsc.subcore_barrier()
    pltpu.sync_copy(cnt_sh.at[pl.ds(sid * ROWS_PER_TILE, ROWS_PER_TILE)], stage_v)
    pltpu.sync_copy(stage_v, out_hbm.at[cid, pl.ds(sid * ROWS_PER_TILE, ROWS_PER_TILE)])


@functools.cache
def _get_sc_count():
    mesh = plsc.VectorSubcoreMesh(core_axis_name="c", subcore_axis_name="s")
    return pl.kernel(
        _sc_count_body,
        out_type=jax.ShapeDtypeStruct((NC, NPAD), jnp.float32),
        mesh=mesh,
        scratch_types=[
            pltpu.VMEM((NCH, CHUNK), jnp.int32),
            pltpu.VMEM((CHUNK,), jnp.float32),
            pltpu.VMEM((ROWS_PER_TILE,), jnp.float32),
            pltpu.VMEM_SHARED((NPAD,), jnp.float32),
            pltpu.SemaphoreType.DMA,
        ],
    )


def _sc_scatter_body(ei_hbm, xs_hbm, out_hbm, src_v, dst_v,
                     r0, acc_sh, g0):
    cid = lax.axis_index("c")
    sid = lax.axis_index("s")
    wid = cid * NS + sid

    _zero_vmem_rows(r0, CHUNK)

    base = sid * ROWS_PER_TILE
    for t in range(ROWS_PER_TILE // CHUNK):
        pltpu.sync_copy(r0, acc_sh.at[pl.ds(base + t * CHUNK, CHUNK)])

    pltpu.sync_copy(ei_hbm.at[0, wid], src_v)
    pltpu.sync_copy(ei_hbm.at[1, wid], dst_v)
    plsc.subcore_barrier()

    @pl.loop(0, NCH)
    def _(j):
        pltpu.async_copy(xs_hbm.at[src_v.at[j]], r0, g0).wait()
        pltpu.sync_copy(r0, acc_sh.at[dst_v.at[j]], add=True)

    plsc.subcore_barrier()
    for t in range(ROWS_PER_TILE // CHUNK):
        r = base + t * CHUNK
        pltpu.sync_copy(acc_sh.at[pl.ds(r, CHUNK)], r0)
        pltpu.sync_copy(r0, out_hbm.at[cid, pl.ds(r, CHUNK)])


@functools.cache
def _get_sc_scatter():
    mesh = plsc.VectorSubcoreMesh(core_axis_name="c", subcore_axis_name="s")
    return pl.kernel(
        _sc_scatter_body,
        out_type=jax.ShapeDtypeStruct((NC, NPAD, D), jnp.float32),
        mesh=mesh,
        scratch_types=[
            pltpu.VMEM((NCH, CHUNK), jnp.int32),
            pltpu.VMEM((NCH, CHUNK), jnp.int32),
            pltpu.VMEM((CHUNK, D), jnp.float32),
            pltpu.VMEM_SHARED((NPAD, D), jnp.float32),
            pltpu.SemaphoreType.DMA,
        ],
    )


def _tc_first_body(cnt_ref, x_ref, w_ref, xs_ref, dinv_ref):
    i = pl.program_id(0)
    cnt = cnt_ref[0, 0, 0, :] + cnt_ref[1, 0, 0, :] + 1.0
    row = i * 128 + lax.broadcasted_iota(jnp.int32, (128,), 0)
    dinv = jnp.where(row < N, lax.rsqrt(cnt), 0.0)
    dinv_ref[0, 0, :] = dinv
    xw = jnp.dot(x_ref[...], w_ref[...], preferred_element_type=jnp.float32)
    xs_ref[...] = xw * dinv[:, None]


def _tc_first(cnt, xpad, W1):
    cnt3 = cnt.reshape(NC, NBLK, 1, 128)
    return pl.pallas_call(
        _tc_first_body,
        grid=(NBLK,),
        in_specs=[
            pl.BlockSpec((NC, 1, 1, 128), lambda i: (0, i, 0, 0)),
            pl.BlockSpec((128, D), lambda i: (i, 0)),
            pl.BlockSpec((D, D), lambda i: (0, 0)),
        ],
        out_specs=[
            pl.BlockSpec((128, D), lambda i: (i, 0)),
            pl.BlockSpec((1, 1, 128), lambda i: (i, 0, 0)),
        ],
        out_shape=[
            jax.ShapeDtypeStruct((NPAD, D), jnp.float32),
            jax.ShapeDtypeStruct((NBLK, 1, 128), jnp.float32),
        ],
    )(cnt3, xpad, W1)


def _tc_mid_body(acc_ref, xs_ref, dinv_ref, b_ref, w_ref, xs2_ref):
    dinv = dinv_ref[0, 0, :][:, None]
    h = acc_ref[0] + acc_ref[1] + xs_ref[...]
    h = jax.nn.relu(h * dinv + b_ref[0, :][None, :])
    xw = jnp.dot(h, w_ref[...], preferred_element_type=jnp.float32)
    xs2_ref[...] = xw * dinv


def _tc_mid(acc, xs, dinv, b, W):
    return pl.pallas_call(
        _tc_mid_body,
        grid=(NBLK,),
        in_specs=[
            pl.BlockSpec((NC, 128, D), lambda i: (0, i, 0)),
            pl.BlockSpec((128, D), lambda i: (i, 0)),
            pl.BlockSpec((1, 1, 128), lambda i: (i, 0, 0)),
            pl.BlockSpec((1, D), lambda i: (0, 0)),
            pl.BlockSpec((D, D), lambda i: (0, 0)),
        ],
        out_specs=pl.BlockSpec((128, D), lambda i: (i, 0)),
        out_shape=jax.ShapeDtypeStruct((NPAD, D), jnp.float32),
    )(acc, xs, dinv, b.reshape(1, D), W)


def _tc_last_body(acc_ref, xs_ref, dinv_ref, b_ref, batch_ref,
                  wl1_ref, bl1_ref, wl2_ref, bl2_ref, out_ref,
                  pool_ref, cnt_ref):
    i = pl.program_id(0)

    @pl.when(i == 0)
    def _():
        pool_ref[...] = jnp.zeros((G, D), jnp.float32)
        cnt_ref[...] = jnp.zeros((G, 128), jnp.float32)

    dinv = dinv_ref[0, 0, :][:, None]
    h = acc_ref[0] + acc_ref[1] + xs_ref[...]
    h = jax.nn.relu(h * dinv + b_ref[0, :][None, :])
    ids = batch_ref[0, 0, :]
    p = (ids[:, None] == lax.broadcasted_iota(jnp.int32, (128, G), 1)).astype(
        jnp.float32
    )
    pool_ref[...] += jnp.dot(p.T, h, preferred_element_type=jnp.float32)
    cnt_ref[...] += jnp.dot(
        p.T, jnp.ones((128, 128), jnp.float32), preferred_element_type=jnp.float32
    )

    @pl.when(i == pl.num_programs(0) - 1)
    def _():
        cnt = jnp.maximum(cnt_ref[:, 0:1], 1.0)
        g = pool_ref[...] / cnt
        g = jax.nn.relu(
            jnp.dot(g, wl1_ref[...], preferred_element_type=jnp.float32)
            + bl1_ref[0, :][None, :]
        )
        out_ref[...] = (
            jnp.dot(g, wl2_ref[...], preferred_element_type=jnp.float32)
            + bl2_ref[0, :][None, :]
        )


def _tc_last(acc, xs, dinv, b, batch3, Wl1, bl1, Wl2p, bl2p):
    return pl.pallas_call(
        _tc_last_body,
        grid=(NBLK,),
        in_specs=[
            pl.BlockSpec((NC, 128, D), lambda i: (0, i, 0)),
            pl.BlockSpec((128, D), lambda i: (i, 0)),
            pl.BlockSpec((1, 1, 128), lambda i: (i, 0, 0)),
            pl.BlockSpec((1, D), lambda i: (0, 0)),
            pl.BlockSpec((1, 1, 128), lambda i: (i, 0, 0)),
            pl.BlockSpec((D, D), lambda i: (0, 0)),
            pl.BlockSpec((1, D), lambda i: (0, 0)),
            pl.BlockSpec((D, 128), lambda i: (0, 0)),
            pl.BlockSpec((1, 128), lambda i: (0, 0)),
        ],
        out_specs=pl.BlockSpec((G, 128), lambda i: (0, 0)),
        out_shape=jax.ShapeDtypeStruct((G, 128), jnp.float32),
        scratch_shapes=[
            pltpu.VMEM((G, D), jnp.float32),
            pltpu.VMEM((G, 128), jnp.float32),
        ],
    )(acc, xs, dinv, b.reshape(1, D), batch3, Wl1, bl1.reshape(1, D), Wl2p, bl2p)


def kernel(x, edge_index, batch, W1, b1, W2, b2, Wl1, bl1, Wl2, bl2):
    xpad = jnp.pad(x, ((0, NPAD - N), (0, 0)))
    ei = edge_index.astype(jnp.int32)
    pad_idx = N + jnp.arange(EPAD - E, dtype=jnp.int32) % (NPAD - N)
    ei = jnp.concatenate([ei, jnp.broadcast_to(pad_idx, (2, EPAD - E))], axis=1)
    ei4 = ei.reshape(2, NW, NCH, CHUNK)
    batch3 = jnp.pad(batch.astype(jnp.int32), (0, NPAD - N), constant_values=-1)
    batch3 = batch3.reshape(NBLK, 1, 128)
    Wl2p = jnp.pad(Wl2, ((0, 0), (0, 128 - N_CLS)))
    bl2p = jnp.pad(bl2, (0, 128 - N_CLS)).reshape(1, 128)

    sc_count = _get_sc_count()
    sc_scatter = _get_sc_scatter()
    cnt = sc_count(ei4)
    xs1, dinv = _tc_first(cnt, xpad, W1)

    acc1 = sc_scatter(ei4, xs1)
    xs2 = _tc_mid(acc1, xs1, dinv, b1, W2)
    acc2 = sc_scatter(ei4, xs2)
    out = _tc_last(acc2, xs2, dinv, b2, batch3, Wl1, bl1, Wl2p, bl2p)
    return out[:, :N_CLS]


# speedup vs baseline: 2.6114x; 1.1014x over previous
import functools

import jax
import jax.numpy as jnp
from jax import lax
from jax.experimental import pallas as pl
from jax.experimental.pallas import tpu as pltpu
from jax.experimental.pallas import tpu_sc as plsc

N = 10000
E = 320000
D = 128
G = 64
N_CLS = 10

NPAD = 10240
NBLK = NPAD // 128
NC = 2
NS = 16
NW = NC * NS
CHUNK = 128
NCH = 80
NBUF = 2
EPW = NCH * CHUNK
EPAD = NW * EPW
ROWS_PER_TILE = NPAD // NS


def _zero_vmem_rows(ref, nrows):
    z = jnp.zeros((16,), jnp.float32)

    @pl.loop(0, nrows)
    def _(i):
        for k in range(8):
            ref[i, pl.ds(k * 16, 16)] = z


def _sc_count_body(ei_hbm, out_hbm, dst_v, ones_v, stage_v, cnt_sh, sem):
    cid = lax.axis_index("c")
    sid = lax.axis_index("s")
    wid = cid * NS + sid

    one = jnp.ones((16,), jnp.float32)
    zero = jnp.zeros((16,), jnp.float32)

    @pl.loop(0, CHUNK // 16)
    def _(i):
        ones_v[pl.ds(i * 16, 16)] = one

    @pl.loop(0, ROWS_PER_TILE // 16)
    def _(i):
        stage_v[pl.ds(i * 16, 16)] = zero

    pltpu.sync_copy(stage_v, cnt_sh.at[pl.ds(sid * ROWS_PER_TILE, ROWS_PER_TILE)])

    pltpu.sync_copy(ei_hbm.at[1, wid], dst_v)
    plsc.subcore_barrier()

    @pl.loop(0, NCH)
    def _(j):
        pltpu.sync_copy(ones_v, cnt_sh.at[dst_v.at[j]], add=True)

    plsc.subcore_barrier()
    pltpu.sync_copy(cnt_sh.at[pl.ds(sid * ROWS_PER_TILE, ROWS_PER_TILE)], stage_v)
    pltpu.sync_copy(stage_v, out_hbm.at[cid, pl.ds(sid * ROWS_PER_TILE, ROWS_PER_TILE)])


@functools.cache
def _get_sc_count():
    mesh = plsc.VectorSubcoreMesh(core_axis_name="c", subcore_axis_name="s")
    return pl.kernel(
        _sc_count_body,
        out_type=jax.ShapeDtypeStruct((NC, NPAD), jnp.float32),
        mesh=mesh,
        scratch_types=[
            pltpu.VMEM((NCH, CHUNK), jnp.int32),
            pltpu.VMEM((CHUNK,), jnp.float32),
            pltpu.VMEM((ROWS_PER_TILE,), jnp.float32),
            pltpu.VMEM_SHARED((NPAD,), jnp.float32),
            pltpu.SemaphoreType.DMA,
        ],
    )


def _sc_scatter_body(ei_hbm, xs_hbm, out_hbm, dst_v, ss0, ss1,
                     r0, r1, acc_sh, g0, g1, i0, i1):
    cid = lax.axis_index("c")
    sid = lax.axis_index("s")
    wid = cid * NS + sid
    sidx = [ss0, ss1]
    bufs = [r0, r1]
    gsems = [g0, g1]
    isems = [i0, i1]

    _zero_vmem_rows(r0, CHUNK)

    base = sid * ROWS_PER_TILE
    for t in range(ROWS_PER_TILE // CHUNK):
        pltpu.sync_copy(r0, acc_sh.at[pl.ds(base + t * CHUNK, CHUNK)])

    pltpu.sync_copy(ei_hbm.at[1, wid], dst_v)
    for b in range(2):
        pltpu.async_copy(ei_hbm.at[0, wid, b], sidx[b], isems[b])
    plsc.subcore_barrier()

    @pl.loop(0, NCH // 2 - 1)
    def _(i):
        for b in range(2):
            j = 2 * i + b
            pltpu.make_async_copy(ei_hbm.at[0, wid, j], sidx[b], isems[b]).wait()
            pltpu.async_copy(xs_hbm.at[sidx[b]], bufs[b], gsems[b])
        for b in range(2):
            j = 2 * i + b
            pltpu.make_async_copy(xs_hbm.at[sidx[b]], bufs[b], gsems[b]).wait()
            pltpu.async_copy(ei_hbm.at[0, wid, j + 2], sidx[b], isems[b])
            pltpu.sync_copy(bufs[b], acc_sh.at[dst_v.at[j]], add=True)

    ilast = NCH - 2
    for b in range(2):
        pltpu.make_async_copy(ei_hbm.at[0, wid, ilast + b], sidx[b], isems[b]).wait()
        pltpu.async_copy(xs_hbm.at[sidx[b]], bufs[b], gsems[b])
    for b in range(2):
        pltpu.make_async_copy(xs_hbm.at[sidx[b]], bufs[b], gsems[b]).wait()
        pltpu.sync_copy(bufs[b], acc_sh.at[dst_v.at[ilast + b]], add=True)

    plsc.subcore_barrier()
    for t in range(ROWS_PER_TILE // CHUNK):
        r = base + t * CHUNK
        pltpu.sync_copy(acc_sh.at[pl.ds(r, CHUNK)], r0)
        pltpu.sync_copy(r0, out_hbm.at[cid, pl.ds(r, CHUNK)])


@functools.cache
def _get_sc_scatter():
    mesh = plsc.VectorSubcoreMesh(core_axis_name="c", subcore_axis_name="s")
    return pl.kernel(
        _sc_scatter_body,
        out_type=jax.ShapeDtypeStruct((NC, NPAD, D), jnp.float32),
        mesh=mesh,
        scratch_types=[
            pltpu.VMEM((NCH, CHUNK), jnp.int32),
            pltpu.VMEM((CHUNK,), jnp.int32),
            pltpu.VMEM((CHUNK,), jnp.int32),
            pltpu.VMEM((CHUNK, D), jnp.float32),
            pltpu.VMEM((CHUNK, D), jnp.float32),
            pltpu.VMEM_SHARED((NPAD, D), jnp.float32),
            pltpu.SemaphoreType.DMA,
            pltpu.SemaphoreType.DMA,
            pltpu.SemaphoreType.DMA,
            pltpu.SemaphoreType.DMA,
        ],
    )


def _tc_first_body(cnt_ref, x_ref, w_ref, xs_ref, dinv_ref):
    i = pl.program_id(0)
    cnt = cnt_ref[0, 0, 0, :] + cnt_ref[1, 0, 0, :] + 1.0
    row = i * 128 + lax.broadcasted_iota(jnp.int32, (128,), 0)
    dinv = jnp.where(row < N, lax.rsqrt(cnt), 0.0)
    dinv_ref[0, 0, :] = dinv
    xw = jnp.dot(x_ref[...], w_ref[...], preferred_element_type=jnp.float32)
    xs_ref[...] = xw * dinv[:, None]


def _tc_first(cnt, xpad, W1):
    cnt3 = cnt.reshape(NC, NBLK, 1, 128)
    return pl.pallas_call(
        _tc_first_body,
        grid=(NBLK,),
        in_specs=[
            pl.BlockSpec((NC, 1, 1, 128), lambda i: (0, i, 0, 0)),
            pl.BlockSpec((128, D), lambda i: (i, 0)),
            pl.BlockSpec((D, D), lambda i: (0, 0)),
        ],
        out_specs=[
            pl.BlockSpec((128, D), lambda i: (i, 0)),
            pl.BlockSpec((1, 1, 128), lambda i: (i, 0, 0)),
        ],
        out_shape=[
            jax.ShapeDtypeStruct((NPAD, D), jnp.float32),
            jax.ShapeDtypeStruct((NBLK, 1, 128), jnp.float32),
        ],
    )(cnt3, xpad, W1)


def _tc_mid_body(acc_ref, xs_ref, dinv_ref, b_ref, w_ref, xs2_ref):
    dinv = dinv_ref[0, 0, :][:, None]
    h = acc_ref[0] + acc_ref[1] + xs_ref[...]
    h = jax.nn.relu(h * dinv + b_ref[0, :][None, :])
    xw = jnp.dot(h, w_ref[...], preferred_element_type=jnp.float32)
    xs2_ref[...] = xw * dinv


def _tc_mid(acc, xs, dinv, b, W):
    return pl.pallas_call(
        _tc_mid_body,
        grid=(NBLK,),
        in_specs=[
            pl.BlockSpec((NC, 128, D), lambda i: (0, i, 0)),
            pl.BlockSpec((128, D), lambda i: (i, 0)),
            pl.BlockSpec((1, 1, 128), lambda i: (i, 0, 0)),
            pl.BlockSpec((1, D), lambda i: (0, 0)),
            pl.BlockSpec((D, D), lambda i: (0, 0)),
        ],
        out_specs=pl.BlockSpec((128, D), lambda i: (i, 0)),
        out_shape=jax.ShapeDtypeStruct((NPAD, D), jnp.float32),
    )(acc, xs, dinv, b.reshape(1, D), W)


def _tc_last_body(acc_ref, xs_ref, dinv_ref, b_ref, batch_ref,
                  wl1_ref, bl1_ref, wl2_ref, bl2_ref, out_ref,
                  pool_ref, cnt_ref):
    i = pl.program_id(0)

    @pl.when(i == 0)
    def _():
        pool_ref[...] = jnp.zeros((G, D), jnp.float32)
        cnt_ref[...] = jnp.zeros((G, 128), jnp.float32)

    dinv = dinv_ref[0, 0, :][:, None]
    h = acc_ref[0] + acc_ref[1] + xs_ref[...]
    h = jax.nn.relu(h * dinv + b_ref[0, :][None, :])
    ids = batch_ref[0, 0, :]
    p = (ids[:, None] == lax.broadcasted_iota(jnp.int32, (128, G), 1)).astype(
        jnp.float32
    )
    pool_ref[...] += jnp.dot(p.T, h, preferred_element_type=jnp.float32)
    cnt_ref[...] += jnp.dot(
        p.T, jnp.ones((128, 128), jnp.float32), preferred_element_type=jnp.float32
    )

    @pl.when(i == pl.num_programs(0) - 1)
    def _():
        cnt = jnp.maximum(cnt_ref[:, 0:1], 1.0)
        g = pool_ref[...] / cnt
        g = jax.nn.relu(
            jnp.dot(g, wl1_ref[...], preferred_element_type=jnp.float32)
            + bl1_ref[0, :][None, :]
        )
        out_ref[...] = (
            jnp.dot(g, wl2_ref[...], preferred_element_type=jnp.float32)
            + bl2_ref[0, :][None, :]
        )


def _tc_last(acc, xs, dinv, b, batch3, Wl1, bl1, Wl2p, bl2p):
    return pl.pallas_call(
        _tc_last_body,
        grid=(NBLK,),
        in_specs=[
            pl.BlockSpec((NC, 128, D), lambda i: (0, i, 0)),
            pl.BlockSpec((128, D), lambda i: (i, 0)),
            pl.BlockSpec((1, 1, 128), lambda i: (i, 0, 0)),
            pl.BlockSpec((1, D), lambda i: (0, 0)),
            pl.BlockSpec((1, 1, 128), lambda i: (i, 0, 0)),
            pl.BlockSpec((D, D), lambda i: (0, 0)),
            pl.BlockSpec((1, D), lambda i: (0, 0)),
            pl.BlockSpec((D, 128), lambda i: (0, 0)),
            pl.BlockSpec((1, 128), lambda i: (0, 0)),
        ],
        out_specs=pl.BlockSpec((G, 128), lambda i: (0, 0)),
        out_shape=jax.ShapeDtypeStruct((G, 128), jnp.float32),
        scratch_shapes=[
            pltpu.VMEM((G, D), jnp.float32),
            pltpu.VMEM((G, 128), jnp.float32),
        ],
    )(acc, xs, dinv, b.reshape(1, D), batch3, Wl1, bl1.reshape(1, D), Wl2p, bl2p)


def kernel(x, edge_index, batch, W1, b1, W2, b2, Wl1, bl1, Wl2, bl2):
    xpad = jnp.pad(x, ((0, NPAD - N), (0, 0)))
    ei = edge_index.astype(jnp.int32)
    pad_idx = N + jnp.arange(EPAD - E, dtype=jnp.int32) % (NPAD - N)
    ei = jnp.concatenate([ei, jnp.broadcast_to(pad_idx, (2, EPAD - E))], axis=1)
    ei4 = ei.reshape(2, NW, NCH, CHUNK)
    batch3 = jnp.pad(batch.astype(jnp.int32), (0, NPAD - N), constant_values=-1)
    batch3 = batch3.reshape(NBLK, 1, 128)
    Wl2p = jnp.pad(Wl2, ((0, 0), (0, 128 - N_CLS)))
    bl2p = jnp.pad(bl2, (0, 128 - N_CLS)).reshape(1, 128)

    sc_count = _get_sc_count()
    sc_scatter = _get_sc_scatter()
    cnt = sc_count(ei4)
    xs1, dinv = _tc_first(cnt, xpad, W1)

    acc1 = sc_scatter(ei4, xs1)
    xs2 = _tc_mid(acc1, xs1, dinv, b1, W2)
    acc2 = sc_scatter(ei4, xs2)
    out = _tc_last(acc2, xs2, dinv, b2, batch3, Wl1, bl1, Wl2p, bl2p)
    return out[:, :N_CLS]


# speedup vs baseline: 2.6754x; 1.0245x over previous
import functools

import jax
import jax.numpy as jnp
from jax import lax
from jax.experimental import pallas as pl
from jax.experimental.pallas import tpu as pltpu
from jax.experimental.pallas import tpu_sc as plsc

N = 10000
E = 320000
D = 128
G = 64
N_CLS = 10

NPAD = 10240
NBLK = NPAD // 128
NC = 2
NS = 16
NW = NC * NS
CHUNK = 128
NCH = 80
NBUF = 2
EPW = NCH * CHUNK
EPAD = NW * EPW
ROWS_PER_TILE = NPAD // NS


def _zero_vmem_rows(ref, nrows):
    z = jnp.zeros((16,), jnp.float32)

    @pl.loop(0, nrows)
    def _(i):
        for k in range(8):
            ref[i, pl.ds(k * 16, 16)] = z


def _sc_count_body(ei_hbm, out_hbm, dst_v, ones_v, stage_v, cnt_sh, sem):
    cid = lax.axis_index("c")
    sid = lax.axis_index("s")
    wid = cid * NS + sid

    one = jnp.ones((16,), jnp.float32)
    zero = jnp.zeros((16,), jnp.float32)

    @pl.loop(0, CHUNK // 16)
    def _(i):
        ones_v[pl.ds(i * 16, 16)] = one

    @pl.loop(0, ROWS_PER_TILE // 16)
    def _(i):
        stage_v[pl.ds(i * 16, 16)] = zero

    pltpu.sync_copy(stage_v, cnt_sh.at[pl.ds(sid * ROWS_PER_TILE, ROWS_PER_TILE)])

    pltpu.sync_copy(ei_hbm.at[1, wid], dst_v)
    plsc.subcore_barrier()

    @pl.loop(0, NCH)
    def _(j):
        pltpu.sync_copy(ones_v, cnt_sh.at[dst_v.at[j]], add=True)

    plsc.subcore_barrier()
    pltpu.sync_copy(cnt_sh.at[pl.ds(sid * ROWS_PER_TILE, ROWS_PER_TILE)], stage_v)
    pltpu.sync_copy(stage_v, out_hbm.at[cid, pl.ds(sid * ROWS_PER_TILE, ROWS_PER_TILE)])


@functools.cache
def _get_sc_count():
    mesh = plsc.VectorSubcoreMesh(core_axis_name="c", subcore_axis_name="s")
    return pl.kernel(
        _sc_count_body,
        out_type=jax.ShapeDtypeStruct((NC, NPAD), jnp.float32),
        mesh=mesh,
        scratch_types=[
            pltpu.VMEM((NCH, CHUNK), jnp.int32),
            pltpu.VMEM((CHUNK,), jnp.float32),
            pltpu.VMEM((ROWS_PER_TILE,), jnp.float32),
            pltpu.VMEM_SHARED((NPAD,), jnp.float32),
            pltpu.SemaphoreType.DMA,
        ],
    )


def _sc_scatter_body(ei_hbm, xs_hbm, out_hbm, dst_v, ss0, ss1,
                     r0, r1, acc_sh, g0, g1, i0, i1, s0, s1):
    cid = lax.axis_index("c")
    sid = lax.axis_index("s")
    wid = cid * NS + sid
    sidx = [ss0, ss1]
    bufs = [r0, r1]
    gsems = [g0, g1]
    isems = [i0, i1]
    ssems = [s0, s1]

    _zero_vmem_rows(r0, CHUNK)

    base = sid * ROWS_PER_TILE
    for t in range(ROWS_PER_TILE // CHUNK):
        pltpu.sync_copy(r0, acc_sh.at[pl.ds(base + t * CHUNK, CHUNK)])

    pltpu.sync_copy(ei_hbm.at[1, wid], dst_v)
    for b in range(2):
        pltpu.async_copy(ei_hbm.at[0, wid, b], sidx[b], isems[b])
    plsc.subcore_barrier()

    for b in range(2):
        pltpu.make_async_copy(ei_hbm.at[0, wid, b], sidx[b], isems[b]).wait()
        pltpu.async_copy(xs_hbm.at[sidx[b]], bufs[b], gsems[b])
    for b in range(2):
        pltpu.make_async_copy(xs_hbm.at[sidx[b]], bufs[b], gsems[b]).wait()
        pltpu.async_copy(ei_hbm.at[0, wid, b + 2], sidx[b], isems[b])
        pltpu.async_copy(bufs[b], acc_sh.at[dst_v.at[b]], ssems[b], add=True)

    @pl.loop(1, NCH // 2 - 1)
    def _(i):
        for b in range(2):
            j = 2 * i + b
            pltpu.make_async_copy(
                bufs[b], acc_sh.at[dst_v.at[j - 2]], ssems[b]
            ).wait()
            pltpu.make_async_copy(ei_hbm.at[0, wid, j], sidx[b], isems[b]).wait()
            pltpu.async_copy(xs_hbm.at[sidx[b]], bufs[b], gsems[b])
        for b in range(2):
            j = 2 * i + b
            pltpu.make_async_copy(xs_hbm.at[sidx[b]], bufs[b], gsems[b]).wait()
            pltpu.async_copy(ei_hbm.at[0, wid, j + 2], sidx[b], isems[b])
            pltpu.async_copy(bufs[b], acc_sh.at[dst_v.at[j]], ssems[b], add=True)

    ilast = NCH - 2
    for b in range(2):
        pltpu.make_async_copy(
            bufs[b], acc_sh.at[dst_v.at[ilast + b - 2]], ssems[b]
        ).wait()
        pltpu.make_async_copy(ei_hbm.at[0, wid, ilast + b], sidx[b], isems[b]).wait()
        pltpu.async_copy(xs_hbm.at[sidx[b]], bufs[b], gsems[b])
    for b in range(2):
        pltpu.make_async_copy(xs_hbm.at[sidx[b]], bufs[b], gsems[b]).wait()
        pltpu.async_copy(bufs[b], acc_sh.at[dst_v.at[ilast + b]], ssems[b], add=True)
    for b in range(2):
        pltpu.make_async_copy(bufs[b], acc_sh.at[dst_v.at[ilast + b]], ssems[b]).wait()

    plsc.subcore_barrier()
    for t in range(ROWS_PER_TILE // CHUNK):
        r = base + t * CHUNK
        pltpu.sync_copy(acc_sh.at[pl.ds(r, CHUNK)], r0)
        pltpu.sync_copy(r0, out_hbm.at[cid, pl.ds(r, CHUNK)])


@functools.cache
def _get_sc_scatter():
    mesh = plsc.VectorSubcoreMesh(core_axis_name="c", subcore_axis_name="s")
    return pl.kernel(
        _sc_scatter_body,
        out_type=jax.ShapeDtypeStruct((NC, NPAD, D), jnp.float32),
        mesh=mesh,
        scratch_types=[
            pltpu.VMEM((NCH, CHUNK), jnp.int32),
            pltpu.VMEM((CHUNK,), jnp.int32),
            pltpu.VMEM((CHUNK,), jnp.int32),
            pltpu.VMEM((CHUNK, D), jnp.float32),
            pltpu.VMEM((CHUNK, D), jnp.float32),
            pltpu.VMEM_SHARED((NPAD, D), jnp.float32),
            pltpu.SemaphoreType.DMA,
            pltpu.SemaphoreType.DMA,
            pltpu.SemaphoreType.DMA,
            pltpu.SemaphoreType.DMA,
            pltpu.SemaphoreType.DMA,
            pltpu.SemaphoreType.DMA,
        ],
    )


def _tc_first_body(cnt_ref, x_ref, w_ref, xs_ref, dinv_ref):
    i = pl.program_id(0)
    cnt = cnt_ref[0, 0, 0, :] + cnt_ref[1, 0, 0, :] + 1.0
    row = i * 128 + lax.broadcasted_iota(jnp.int32, (128,), 0)
    dinv = jnp.where(row < N, lax.rsqrt(cnt), 0.0)
    dinv_ref[0, 0, :] = dinv
    xw = jnp.dot(x_ref[...], w_ref[...], preferred_element_type=jnp.float32)
    xs_ref[...] = xw * dinv[:, None]


def _tc_first(cnt, xpad, W1):
    cnt3 = cnt.reshape(NC, NBLK, 1, 128)
    return pl.pallas_call(
        _tc_first_body,
        grid=(NBLK,),
        in_specs=[
            pl.BlockSpec((NC, 1, 1, 128), lambda i: (0, i, 0, 0)),
            pl.BlockSpec((128, D), lambda i: (i, 0)),
            pl.BlockSpec((D, D), lambda i: (0, 0)),
        ],
        out_specs=[
            pl.BlockSpec((128, D), lambda i: (i, 0)),
            pl.BlockSpec((1, 1, 128), lambda i: (i, 0, 0)),
        ],
        out_shape=[
            jax.ShapeDtypeStruct((NPAD, D), jnp.float32),
            jax.ShapeDtypeStruct((NBLK, 1, 128), jnp.float32),
        ],
    )(cnt3, xpad, W1)


def _tc_mid_body(acc_ref, xs_ref, dinv_ref, b_ref, w_ref, xs2_ref):
    dinv = dinv_ref[0, 0, :][:, None]
    h = acc_ref[0] + acc_ref[1] + xs_ref[...]
    h = jax.nn.relu(h * dinv + b_ref[0, :][None, :])
    xw = jnp.dot(h, w_ref[...], preferred_element_type=jnp.float32)
    xs2_ref[...] = xw * dinv


def _tc_mid(acc, xs, dinv, b, W):
    return pl.pallas_call(
        _tc_mid_body,
        grid=(NBLK,),
        in_specs=[
            pl.BlockSpec((NC, 128, D), lambda i: (0, i, 0)),
            pl.BlockSpec((128, D), lambda i: (i, 0)),
            pl.BlockSpec((1, 1, 128), lambda i: (i, 0, 0)),
            pl.BlockSpec((1, D), lambda i: (0, 0)),
            pl.BlockSpec((D, D), lambda i: (0, 0)),
        ],
        out_specs=pl.BlockSpec((128, D), lambda i: (i, 0)),
        out_shape=jax.ShapeDtypeStruct((NPAD, D), jnp.float32),
    )(acc, xs, dinv, b.reshape(1, D), W)


def _tc_last_body(acc_ref, xs_ref, dinv_ref, b_ref, batch_ref,
                  wl1_ref, bl1_ref, wl2_ref, bl2_ref, out_ref,
                  pool_ref, cnt_ref):
    i = pl.program_id(0)

    @pl.when(i == 0)
    def _():
        pool_ref[...] = jnp.zeros((G, D), jnp.float32)
        cnt_ref[...] = jnp.zeros((G, 128), jnp.float32)

    dinv = dinv_ref[0, 0, :][:, None]
    h = acc_ref[0] + acc_ref[1] + xs_ref[...]
    h = jax.nn.relu(h * dinv + b_ref[0, :][None, :])
    ids = batch_ref[0, 0, :]
    p = (ids[:, None] == lax.broadcasted_iota(jnp.int32, (128, G), 1)).astype(
        jnp.float32
    )
    pool_ref[...] += jnp.dot(p.T, h, preferred_element_type=jnp.float32)
    cnt_ref[...] += jnp.dot(
        p.T, jnp.ones((128, 128), jnp.float32), preferred_element_type=jnp.float32
    )

    @pl.when(i == pl.num_programs(0) - 1)
    def _():
        cnt = jnp.maximum(cnt_ref[:, 0:1], 1.0)
        g = pool_ref[...] / cnt
        g = jax.nn.relu(
            jnp.dot(g, wl1_ref[...], preferred_element_type=jnp.float32)
            + bl1_ref[0, :][None, :]
        )
        out_ref[...] = (
            jnp.dot(g, wl2_ref[...], preferred_element_type=jnp.float32)
            + bl2_ref[0, :][None, :]
        )


def _tc_last(acc, xs, dinv, b, batch3, Wl1, bl1, Wl2p, bl2p):
    return pl.pallas_call(
        _tc_last_body,
        grid=(NBLK,),
        in_specs=[
            pl.BlockSpec((NC, 128, D), lambda i: (0, i, 0)),
            pl.BlockSpec((128, D), lambda i: (i, 0)),
            pl.BlockSpec((1, 1, 128), lambda i: (i, 0, 0)),
            pl.BlockSpec((1, D), lambda i: (0, 0)),
            pl.BlockSpec((1, 1, 128), lambda i: (i, 0, 0)),
            pl.BlockSpec((D, D), lambda i: (0, 0)),
            pl.BlockSpec((1, D), lambda i: (0, 0)),
            pl.BlockSpec((D, 128), lambda i: (0, 0)),
            pl.BlockSpec((1, 128), lambda i: (0, 0)),
        ],
        out_specs=pl.BlockSpec((G, 128), lambda i: (0, 0)),
        out_shape=jax.ShapeDtypeStruct((G, 128), jnp.float32),
        scratch_shapes=[
            pltpu.VMEM((G, D), jnp.float32),
            pltpu.VMEM((G, 128), jnp.float32),
        ],
    )(acc, xs, dinv, b.reshape(1, D), batch3, Wl1, bl1.reshape(1, D), Wl2p, bl2p)


def kernel(x, edge_index, batch, W1, b1, W2, b2, Wl1, bl1, Wl2, bl2):
    xpad = jnp.pad(x, ((0, NPAD - N), (0, 0)))
    ei = edge_index.astype(jnp.int32)
    pad_idx = N + jnp.arange(EPAD - E, dtype=jnp.int32) % (NPAD - N)
    ei = jnp.concatenate([ei, jnp.broadcast_to(pad_idx, (2, EPAD - E))], axis=1)
    ei4 = ei.reshape(2, NW, NCH, CHUNK)
    batch3 = jnp.pad(batch.astype(jnp.int32), (0, NPAD - N), constant_values=-1)
    batch3 = batch3.reshape(NBLK, 1, 128)
    Wl2p = jnp.pad(Wl2, ((0, 0), (0, 128 - N_CLS)))
    bl2p = jnp.pad(bl2, (0, 128 - N_CLS)).reshape(1, 128)

    sc_count = _get_sc_count()
    sc_scatter = _get_sc_scatter()
    cnt = sc_count(ei4)
    xs1, dinv = _tc_first(cnt, xpad, W1)

    acc1 = sc_scatter(ei4, xs1)
    xs2 = _tc_mid(acc1, xs1, dinv, b1, W2)
    acc2 = sc_scatter(ei4, xs2)
    out = _tc_last(acc2, xs2, dinv, b2, batch3, Wl1, bl1, Wl2p, bl2p)
    return out[:, :N_CLS]


# speedup vs baseline: 2.7011x; 1.0096x over previous
import functools

import jax
import jax.numpy as jnp
from jax import lax
from jax.experimental import pallas as pl
from jax.experimental.pallas import tpu as pltpu
from jax.experimental.pallas import tpu_sc as plsc

N = 10000
E = 320000
D = 128
G = 64
N_CLS = 10

NPAD = 10240
NBLK = NPAD // 128
NC = 2
NS = 16
NW = NC * NS
CHUNK = 128
NCH = 80
NBUF = 2
EPW = NCH * CHUNK
EPAD = NW * EPW
ROWS_PER_TILE = NPAD // NS


def _zero_vmem_rows(ref, nrows):
    z = jnp.zeros((16,), jnp.float32)

    @pl.loop(0, nrows)
    def _(i):
        for k in range(8):
            ref[i, pl.ds(k * 16, 16)] = z


def _sc_count_body(ei_hbm, out_hbm, dst_v, ones_v, stage_v, cnt_sh, sem):
    cid = lax.axis_index("c")
    sid = lax.axis_index("s")
    wid = cid * NS + sid

    one = jnp.ones((16,), jnp.float32)
    zero = jnp.zeros((16,), jnp.float32)

    @pl.loop(0, CHUNK // 16)
    def _(i):
        ones_v[pl.ds(i * 16, 16)] = one

    @pl.loop(0, ROWS_PER_TILE // 16)
    def _(i):
        stage_v[pl.ds(i * 16, 16)] = zero

    pltpu.sync_copy(stage_v, cnt_sh.at[pl.ds(sid * ROWS_PER_TILE, ROWS_PER_TILE)])

    pltpu.sync_copy(ei_hbm.at[1, wid], dst_v)
    plsc.subcore_barrier()

    @pl.loop(0, NCH)
    def _(j):
        pltpu.sync_copy(ones_v, cnt_sh.at[dst_v.at[j]], add=True)

    plsc.subcore_barrier()
    pltpu.sync_copy(cnt_sh.at[pl.ds(sid * ROWS_PER_TILE, ROWS_PER_TILE)], stage_v)
    pltpu.sync_copy(stage_v, out_hbm.at[cid, pl.ds(sid * ROWS_PER_TILE, ROWS_PER_TILE)])


@functools.cache
def _get_sc_count():
    mesh = plsc.VectorSubcoreMesh(core_axis_name="c", subcore_axis_name="s")
    return pl.kernel(
        _sc_count_body,
        out_type=jax.ShapeDtypeStruct((NC, NPAD), jnp.float32),
        mesh=mesh,
        scratch_types=[
            pltpu.VMEM((NCH, CHUNK), jnp.int32),
            pltpu.VMEM((CHUNK,), jnp.float32),
            pltpu.VMEM((ROWS_PER_TILE,), jnp.float32),
            pltpu.VMEM_SHARED((NPAD,), jnp.float32),
            pltpu.SemaphoreType.DMA,
        ],
    )


def _sc_scatter_body(ei_hbm, xs_hbm, out_hbm, dst_v, ss0, ss1,
                     r0, r1, acc_sh, g0, g1, i0, i1, s0, s1):
    cid = lax.axis_index("c")
    sid = lax.axis_index("s")
    wid = cid * NS + sid
    sidx = [ss0, ss1]
    bufs = [r0, r1]
    gsems = [g0, g1]
    isems = [i0, i1]
    ssems = [s0, s1]

    _zero_vmem_rows(r0, CHUNK)

    base = sid * ROWS_PER_TILE
    for t in range(ROWS_PER_TILE // CHUNK):
        pltpu.async_copy(r0, acc_sh.at[pl.ds(base + t * CHUNK, CHUNK)], s0)

    pltpu.sync_copy(ei_hbm.at[1, wid], dst_v)
    for b in range(2):
        pltpu.async_copy(ei_hbm.at[0, wid, b], sidx[b], isems[b])
    for t in range(ROWS_PER_TILE // CHUNK):
        pltpu.make_async_copy(r0, acc_sh.at[pl.ds(base + t * CHUNK, CHUNK)], s0).wait()
    plsc.subcore_barrier()

    for b in range(2):
        pltpu.make_async_copy(ei_hbm.at[0, wid, b], sidx[b], isems[b]).wait()
        pltpu.async_copy(xs_hbm.at[sidx[b]], bufs[b], gsems[b])
    for b in range(2):
        pltpu.make_async_copy(xs_hbm.at[sidx[b]], bufs[b], gsems[b]).wait()
        pltpu.async_copy(ei_hbm.at[0, wid, b + 2], sidx[b], isems[b])
        pltpu.async_copy(bufs[b], acc_sh.at[dst_v.at[b]], ssems[b], add=True)

    @pl.loop(1, NCH // 2 - 1)
    def _(i):
        for b in range(2):
            j = 2 * i + b
            pltpu.make_async_copy(
                bufs[b], acc_sh.at[dst_v.at[j - 2]], ssems[b]
            ).wait()
            pltpu.make_async_copy(ei_hbm.at[0, wid, j], sidx[b], isems[b]).wait()
            pltpu.async_copy(xs_hbm.at[sidx[b]], bufs[b], gsems[b])
        for b in range(2):
            j = 2 * i + b
            pltpu.make_async_copy(xs_hbm.at[sidx[b]], bufs[b], gsems[b]).wait()
            pltpu.async_copy(ei_hbm.at[0, wid, j + 2], sidx[b], isems[b])
            pltpu.async_copy(bufs[b], acc_sh.at[dst_v.at[j]], ssems[b], add=True)

    ilast = NCH - 2
    for b in range(2):
        pltpu.make_async_copy(
            bufs[b], acc_sh.at[dst_v.at[ilast + b - 2]], ssems[b]
        ).wait()
        pltpu.make_async_copy(ei_hbm.at[0, wid, ilast + b], sidx[b], isems[b]).wait()
        pltpu.async_copy(xs_hbm.at[sidx[b]], bufs[b], gsems[b])
    for b in range(2):
        pltpu.make_async_copy(xs_hbm.at[sidx[b]], bufs[b], gsems[b]).wait()
        pltpu.async_copy(bufs[b], acc_sh.at[dst_v.at[ilast + b]], ssems[b], add=True)
    for b in range(2):
        pltpu.make_async_copy(bufs[b], acc_sh.at[dst_v.at[ilast + b]], ssems[b]).wait()

    plsc.subcore_barrier()
    pltpu.sync_copy(
        acc_sh.at[pl.ds(base, ROWS_PER_TILE)],
        out_hbm.at[cid, pl.ds(base, ROWS_PER_TILE)],
    )


@functools.cache
def _get_sc_scatter():
    mesh = plsc.VectorSubcoreMesh(core_axis_name="c", subcore_axis_name="s")
    return pl.kernel(
        _sc_scatter_body,
        out_type=jax.ShapeDtypeStruct((NC, NPAD, D), jnp.float32),
        mesh=mesh,
        scratch_types=[
            pltpu.VMEM((NCH, CHUNK), jnp.int32),
            pltpu.VMEM((CHUNK,), jnp.int32),
            pltpu.VMEM((CHUNK,), jnp.int32),
            pltpu.VMEM((CHUNK, D), jnp.float32),
            pltpu.VMEM((CHUNK, D), jnp.float32),
            pltpu.VMEM_SHARED((NPAD, D), jnp.float32),
            pltpu.SemaphoreType.DMA,
            pltpu.SemaphoreType.DMA,
            pltpu.SemaphoreType.DMA,
            pltpu.SemaphoreType.DMA,
            pltpu.SemaphoreType.DMA,
            pltpu.SemaphoreType.DMA,
        ],
    )


def _tc_first_body(cnt_ref, x_ref, w_ref, xs_ref, dinv_ref):
    i = pl.program_id(0)
    cnt = cnt_ref[0, 0, 0, :] + cnt_ref[1, 0, 0, :] + 1.0
    row = i * 128 + lax.broadcasted_iota(jnp.int32, (128,), 0)
    dinv = jnp.where(row < N, lax.rsqrt(cnt), 0.0)
    dinv_ref[0, 0, :] = dinv
    xw = jnp.dot(x_ref[...], w_ref[...], preferred_element_type=jnp.float32)
    xs_ref[...] = xw * dinv[:, None]


def _tc_first(cnt, xpad, W1):
    cnt3 = cnt.reshape(NC, NBLK, 1, 128)
    return pl.pallas_call(
        _tc_first_body,
        grid=(NBLK,),
        in_specs=[
            pl.BlockSpec((NC, 1, 1, 128), lambda i: (0, i, 0, 0)),
            pl.BlockSpec((128, D), lambda i: (i, 0)),
            pl.BlockSpec((D, D), lambda i: (0, 0)),
        ],
        out_specs=[
            pl.BlockSpec((128, D), lambda i: (i, 0)),
            pl.BlockSpec((1, 1, 128), lambda i: (i, 0, 0)),
        ],
        out_shape=[
            jax.ShapeDtypeStruct((NPAD, D), jnp.float32),
            jax.ShapeDtypeStruct((NBLK, 1, 128), jnp.float32),
        ],
    )(cnt3, xpad, W1)


def _tc_mid_body(acc_ref, xs_ref, dinv_ref, b_ref, w_ref, xs2_ref):
    dinv = dinv_ref[0, 0, :][:, None]
    h = acc_ref[0] + acc_ref[1] + xs_ref[...]
    h = jax.nn.relu(h * dinv + b_ref[0, :][None, :])
    xw = jnp.dot(h, w_ref[...], preferred_element_type=jnp.float32)
    xs2_ref[...] = xw * dinv


def _tc_mid(acc, xs, dinv, b, W):
    return pl.pallas_call(
        _tc_mid_body,
        grid=(NBLK,),
        in_specs=[
            pl.BlockSpec((NC, 128, D), lambda i: (0, i, 0)),
            pl.BlockSpec((128, D), lambda i: (i, 0)),
            pl.BlockSpec((1, 1, 128), lambda i: (i, 0, 0)),
            pl.BlockSpec((1, D), lambda i: (0, 0)),
            pl.BlockSpec((D, D), lambda i: (0, 0)),
        ],
        out_specs=pl.BlockSpec((128, D), lambda i: (i, 0)),
        out_shape=jax.ShapeDtypeStruct((NPAD, D), jnp.float32),
    )(acc, xs, dinv, b.reshape(1, D), W)


def _tc_last_body(acc_ref, xs_ref, dinv_ref, b_ref, batch_ref,
                  wl1_ref, bl1_ref, wl2_ref, bl2_ref, out_ref,
                  pool_ref, cnt_ref):
    i = pl.program_id(0)

    @pl.when(i == 0)
    def _():
        pool_ref[...] = jnp.zeros((G, D), jnp.float32)
        cnt_ref[...] = jnp.zeros((G, 128), jnp.float32)

    dinv = dinv_ref[0, 0, :][:, None]
    h = acc_ref[0] + acc_ref[1] + xs_ref[...]
    h = jax.nn.relu(h * dinv + b_ref[0, :][None, :])
    ids = batch_ref[0, 0, :]
    p = (ids[:, None] == lax.broadcasted_iota(jnp.int32, (128, G), 1)).astype(
        jnp.float32
    )
    pool_ref[...] += jnp.dot(p.T, h, preferred_element_type=jnp.float32)
    cnt_ref[...] += jnp.dot(
        p.T, jnp.ones((128, 128), jnp.float32), preferred_element_type=jnp.float32
    )

    @pl.when(i == pl.num_programs(0) - 1)
    def _():
        cnt = jnp.maximum(cnt_ref[:, 0:1], 1.0)
        g = pool_ref[...] / cnt
        g = jax.nn.relu(
            jnp.dot(g, wl1_ref[...], preferred_element_type=jnp.float32)
            + bl1_ref[0, :][None, :]
        )
        out_ref[...] = (
            jnp.dot(g, wl2_ref[...], preferred_element_type=jnp.float32)
            + bl2_ref[0, :][None, :]
        )


def _tc_last(acc, xs, dinv, b, batch3, Wl1, bl1, Wl2p, bl2p):
    return pl.pallas_call(
        _tc_last_body,
        grid=(NBLK,),
        in_specs=[
            pl.BlockSpec((NC, 128, D), lambda i: (0, i, 0)),
            pl.BlockSpec((128, D), lambda i: (i, 0)),
            pl.BlockSpec((1, 1, 128), lambda i: (i, 0, 0)),
            pl.BlockSpec((1, D), lambda i: (0, 0)),
            pl.BlockSpec((1, 1, 128), lambda i: (i, 0, 0)),
            pl.BlockSpec((D, D), lambda i: (0, 0)),
            pl.BlockSpec((1, D), lambda i: (0, 0)),
            pl.BlockSpec((D, 128), lambda i: (0, 0)),
            pl.BlockSpec((1, 128), lambda i: (0, 0)),
        ],
        out_specs=pl.BlockSpec((G, 128), lambda i: (0, 0)),
        out_shape=jax.ShapeDtypeStruct((G, 128), jnp.float32),
        scratch_shapes=[
            pltpu.VMEM((G, D), jnp.float32),
            pltpu.VMEM((G, 128), jnp.float32),
        ],
    )(acc, xs, dinv, b.reshape(1, D), batch3, Wl1, bl1.reshape(1, D), Wl2p, bl2p)


def kernel(x, edge_index, batch, W1, b1, W2, b2, Wl1, bl1, Wl2, bl2):
    xpad = jnp.pad(x, ((0, NPAD - N), (0, 0)))
    ei = edge_index.astype(jnp.int32)
    pad_idx = N + jnp.arange(EPAD - E, dtype=jnp.int32) % (NPAD - N)
    ei = jnp.concatenate([ei, jnp.broadcast_to(pad_idx, (2, EPAD - E))], axis=1)
    ei4 = ei.reshape(2, NW, NCH, CHUNK)
    batch3 = jnp.pad(batch.astype(jnp.int32), (0, NPAD - N), constant_values=-1)
    batch3 = batch3.reshape(NBLK, 1, 128)
    Wl2p = jnp.pad(Wl2, ((0, 0), (0, 128 - N_CLS)))
    bl2p = jnp.pad(bl2, (0, 128 - N_CLS)).reshape(1, 128)

    sc_count = _get_sc_count()
    sc_scatter = _get_sc_scatter()
    cnt = sc_count(ei4)
    xs1, dinv = _tc_first(cnt, xpad, W1)

    acc1 = sc_scatter(ei4, xs1)
    xs2 = _tc_mid(acc1, xs1, dinv, b1, W2)
    acc2 = sc_scatter(ei4, xs2)
    out = _tc_last(acc2, xs2, dinv, b2, batch3, Wl1, bl1, Wl2p, bl2p)
    return out[:, :N_CLS]
